# baseline (speedup 1.0000x reference)
import functools

import jax
import jax.numpy as jnp
from jax import lax
from jax.experimental import pallas as pl
from jax.experimental.pallas import tpu as pltpu
from jax.experimental.pallas import tpu_sc as plsc

_HC = 8
_NBUF = 8
_OBUF = 8
_B_SC = 32


def _gather_tables(ids, cos_ref, sin_ref):
    n_tok = ids.shape[0]
    n_rows, d = cos_ref.shape
    row_iota = jax.lax.broadcasted_iota(jnp.int32, (n_tok, n_rows), 1)
    onehot = (ids[:, None] == row_iota).astype(jnp.bfloat16)
    lane = jax.lax.broadcasted_iota(jnp.int32, (n_rows, d), 1)
    sin_tab = jnp.where(lane % 2 == 0, -sin_ref[...], sin_ref[...])
    cos_g = jnp.dot(onehot, cos_ref[...].astype(jnp.bfloat16),
                    preferred_element_type=jnp.float32)
    sin_g = jnp.dot(onehot, sin_tab.astype(jnp.bfloat16),
                    preferred_element_type=jnp.float32)
    return cos_g, sin_g


def _rotate_combine(tb, cos_g, sin_g):
    h, n_tok, d = tb.shape
    rowm = jax.lax.broadcasted_iota(jnp.int32, (d, d), 0)
    colm = jax.lax.broadcasted_iota(jnp.int32, (d, d), 1)
    m = ((rowm ^ 1) == colm).astype(jnp.bfloat16)
    t2 = tb.reshape(h * n_tok, d).astype(jnp.bfloat16)
    swap = jnp.dot(t2, m, preferred_element_type=jnp.float32).reshape(h, n_tok, d)
    return tb * cos_g[None] + swap * sin_g[None]


def _rope_manual(ids_ref, cos_ref, sin_ref, t_hbm, out_hbm,
                 in_buf, out_buf, in_sems, out_sems, b_lo=0):
    nb, h, n_tok, d = out_hbm.shape
    cpb = h // _HC
    nchunks = nb * cpb

    def in_dma(c, slot):
        b = c // cpb
        hc = c % cpb
        return pltpu.make_async_copy(
            t_hbm.at[b_lo + b, pl.ds(hc * _HC, _HC)], in_buf.at[slot],
            in_sems.at[slot])

    def out_dma(c, slot):
        b = c // cpb
        hc = c % cpb
        return pltpu.make_async_copy(
            out_buf.at[slot], out_hbm.at[b, pl.ds(hc * _HC, _HC)],
            out_sems.at[slot])

    for c in range(_NBUF):
        in_dma(c, c).start()

    def body(c, _):
        slot = jax.lax.rem(c, _NBUF)
        oslot = jax.lax.rem(c, _OBUF)
        b = c // cpb
        in_dma(c, slot).wait()
        ids = ids_ref[b_lo + b, 0, :]
        cos_g, sin_g = _gather_tables(ids, cos_ref, sin_ref)
        res = _rotate_combine(in_buf[slot], cos_g, sin_g)

        @pl.when(c >= _OBUF)
        def _():
            out_dma(c - _OBUF, oslot).wait()

        out_buf[oslot] = res
        out_dma(c, oslot).start()

        @pl.when(c + _NBUF < nchunks)
        def _():
            in_dma(c + _NBUF, slot).start()

        return _

    jax.lax.fori_loop(0, nchunks, body, None)
    for k in range(_OBUF):
        c = nchunks - _OBUF + k
        out_dma(c, c % _OBUF).wait()


def _make_sc_rope(b_lo, b_sc, h, n, d):
    info = plsc.get_sparse_core_info()
    nw = info.num_cores * info.num_subcores
    tg_n = 8
    hg_n = nw // tg_n
    ntok = n // tg_n
    nh = h // hg_n
    mesh = plsc.VectorSubcoreMesh(core_axis_name="c", subcore_axis_name="s")

    @functools.partial(
        pl.kernel, mesh=mesh,
        out_type=jax.ShapeDtypeStruct((b_sc, h, n, d), jnp.float32),
        scratch_types=[
            pltpu.VMEM((ntok,), jnp.int32),
            pltpu.VMEM((ntok, 2 * d), jnp.float32),
            pltpu.VMEM((ntok, 2 * d), jnp.float32),
            pltpu.VMEM((nh, ntok, d), jnp.float32),
            pltpu.VMEM((nh, ntok, d), jnp.float32),
            pltpu.SemaphoreType.DMA,
            pltpu.SemaphoreType.DMA,
        ],
    )
    def sc_rope(ids_hbm, cos_hbm, sin_hbm, t_hbm, out_hbm,
                idx_v, cosr, sinr, t_v, o_v, sem_c, sem_s):
        wid = lax.axis_index("s") * info.num_cores + lax.axis_index("c")
        n0 = pl.multiple_of(lax.rem(wid, tg_n) * ntok, 8)
        h0 = (wid // tg_n) * nh

        @pl.loop(0, b_sc)
        def _batch(bi):
            lane = lax.iota(jnp.int32, 16)
            par = lax.rem(lane, 2)
            swp = lane + 1 - 2 * par
            sign = (2 * par - 1).astype(jnp.float32)
            b = b_lo + bi
            pltpu.sync_copy(
                ids_hbm.at[pl.ds(pl.multiple_of(b * n + n0, 8), ntok)],
                idx_v)
            cdma = pltpu.async_copy(cos_hbm.at[idx_v], cosr, sem_c)
            sdma = pltpu.async_copy(sin_hbm.at[idx_v], sinr, sem_s)
            pltpu.sync_copy(
                t_hbm.at[b, pl.ds(h0, nh), pl.ds(n0, ntok), :], t_v)
            cdma.wait()
            sdma.wait()

            @pl.loop(0, ntok)
            def _token(nl):
                for k in range(d // 16):
                    cp = cosr[nl, pl.ds(k * 16, 16)]
                    sp = sinr[nl, pl.ds(k * 16, 16)] * sign
                    for hh in range(nh):
                        tt = t_v[hh, nl, pl.ds(k * 16, 16)]
                        sw = lax.gather(
                            tt, swp[:, None],
                            lax.GatherDimensionNumbers(
                                offset_dims=(),
                                collapsed_slice_dims=(0,),
                                start_index_map=(0,)),
                            (1,),
                            mode=lax.GatherScatterMode.PROMISE_IN_BOUNDS)
                        o_v[hh, nl, pl.ds(k * 16, 16)] = tt * cp + sw * sp

            pltpu.sync_copy(
                o_v, out_hbm.at[bi, pl.ds(h0, nh), pl.ds(n0, ntok), :])

    return sc_rope


def kernel(t, rope_ids, freqs_cos, freqs_sin):
    b, h, n, d = t.shape
    b_tc = b - _B_SC
    pad = ((0, 0), (0, d))
    out_sc = _make_sc_rope(b_tc, _B_SC, h, n, d)(
        rope_ids.reshape(-1), jnp.pad(freqs_cos, pad), jnp.pad(freqs_sin, pad), t)
    out_tc = _run_tc(t, rope_ids, freqs_cos, freqs_sin, 0, b_tc)
    return jnp.concatenate([out_tc, out_sc], axis=0)


def _run_tc(t, rope_ids, freqs_cos, freqs_sin, b_lo, b_hi):
    b, h, n, d = t.shape
    nb = b_hi - b_lo
    r = freqs_cos.shape[0]
    ids3 = rope_ids.reshape(b, 1, n)
    return pl.pallas_call(
        lambda *refs: _rope_manual(*refs, b_lo=b_lo),
        in_specs=[
            pl.BlockSpec(memory_space=pltpu.MemorySpace.VMEM),
            pl.BlockSpec(memory_space=pltpu.MemorySpace.VMEM),
            pl.BlockSpec(memory_space=pltpu.MemorySpace.VMEM),
            pl.BlockSpec(memory_space=pltpu.MemorySpace.HBM),
        ],
        out_specs=pl.BlockSpec(memory_space=pltpu.MemorySpace.HBM),
        out_shape=jax.ShapeDtypeStruct((nb, h, n, d), t.dtype),
        scratch_shapes=[
            pltpu.VMEM((_NBUF, _HC, n, d), jnp.float32),
            pltpu.VMEM((_OBUF, _HC, n, d), jnp.float32),
            pltpu.SemaphoreType.DMA((_NBUF,)),
            pltpu.SemaphoreType.DMA((_OBUF,)),
        ],
    )(ids3, freqs_cos, freqs_sin, t)


# speedup vs baseline: 1.3116x; 1.3116x over previous
import jax
import jax.numpy as jnp
from jax.experimental import pallas as pl
from jax.experimental.pallas import tpu as pltpu

_HC = 8
_NBUF = 8
_OBUF = 8


def _gather_tables(ids, cos_ref, sin_ref):
    n_tok = ids.shape[0]
    n_rows, d = cos_ref.shape
    row_iota = jax.lax.broadcasted_iota(jnp.int32, (n_tok, n_rows), 1)
    onehot = (ids[:, None] == row_iota).astype(jnp.bfloat16)
    lane = jax.lax.broadcasted_iota(jnp.int32, (n_rows, d), 1)
    sin_tab = jnp.where(lane % 2 == 0, -sin_ref[...], sin_ref[...])
    cos_g = jnp.dot(onehot, cos_ref[...].astype(jnp.bfloat16),
                    preferred_element_type=jnp.float32)
    sin_g = jnp.dot(onehot, sin_tab.astype(jnp.bfloat16),
                    preferred_element_type=jnp.float32)
    return cos_g, sin_g


def _rotate_combine(tb, cos_g, sin_g):
    h, n_tok, d = tb.shape
    rowm = jax.lax.broadcasted_iota(jnp.int32, (d, d), 0)
    colm = jax.lax.broadcasted_iota(jnp.int32, (d, d), 1)
    m = ((rowm ^ 1) == colm).astype(jnp.bfloat16)
    t2 = tb.reshape(h * n_tok, d).astype(jnp.bfloat16)
    swap = jnp.dot(t2, m, preferred_element_type=jnp.float32).reshape(h, n_tok, d)
    return tb * cos_g[None] + swap * sin_g[None]


def _rope_manual(ids_ref, cos_ref, sin_ref, t_hbm, out_hbm,
                 in_buf, out_buf, in_sems, out_sems):
    b_total, h, n_tok, d = t_hbm.shape
    cpb = h // _HC
    nchunks = b_total * cpb

    def in_dma(c, slot):
        b = c // cpb
        hc = c % cpb
        return pltpu.make_async_copy(
            t_hbm.at[b, pl.ds(hc * _HC, _HC)], in_buf.at[slot],
            in_sems.at[slot])

    def out_dma(c, slot):
        b = c // cpb
        hc = c % cpb
        return pltpu.make_async_copy(
            out_buf.at[slot], out_hbm.at[b, pl.ds(hc * _HC, _HC)],
            out_sems.at[slot])

    for c in range(_NBUF):
        in_dma(c, c).start()

    def body(c, _):
        slot = jax.lax.rem(c, _NBUF)
        oslot = jax.lax.rem(c, _OBUF)
        b = c // cpb
        in_dma(c, slot).wait()
        ids = ids_ref[b, 0, :]
        cos_g, sin_g = _gather_tables(ids, cos_ref, sin_ref)
        res = _rotate_combine(in_buf[slot], cos_g, sin_g)

        @pl.when(c >= _OBUF)
        def _():
            out_dma(c - _OBUF, oslot).wait()

        out_buf[oslot] = res
        out_dma(c, oslot).start()

        @pl.when(c + _NBUF < nchunks)
        def _():
            in_dma(c + _NBUF, slot).start()

        return _

    jax.lax.fori_loop(0, nchunks, body, None)
    for k in range(_OBUF):
        c = nchunks - _OBUF + k
        out_dma(c, c % _OBUF).wait()


def kernel(t, rope_ids, freqs_cos, freqs_sin):
    b, h, n, d = t.shape
    r = freqs_cos.shape[0]
    ids3 = rope_ids.reshape(b, 1, n)
    return pl.pallas_call(
        _rope_manual,
        in_specs=[
            pl.BlockSpec(memory_space=pltpu.MemorySpace.VMEM),
            pl.BlockSpec(memory_space=pltpu.MemorySpace.VMEM),
            pl.BlockSpec(memory_space=pltpu.MemorySpace.VMEM),
            pl.BlockSpec(memory_space=pltpu.MemorySpace.HBM),
        ],
        out_specs=pl.BlockSpec(memory_space=pltpu.MemorySpace.HBM),
        out_shape=jax.ShapeDtypeStruct((b, h, n, d), t.dtype),
        scratch_shapes=[
            pltpu.VMEM((_NBUF, _HC, n, d), jnp.float32),
            pltpu.VMEM((_OBUF, _HC, n, d), jnp.float32),
            pltpu.SemaphoreType.DMA((_NBUF,)),
            pltpu.SemaphoreType.DMA((_OBUF,)),
        ],
    )(ids3, freqs_cos, freqs_sin, t)
